# baseline (speedup 1.0000x reference)
import jax
import jax.numpy as jnp
from jax.experimental import pallas as pl
from jax.experimental.pallas import tpu as pltpu

N_TOK = 16384
E_DIM = 32
N_E = 1024
NUM_Q = 4
MU = 0.25
TILE = 256


def _rvq_kernel(x_ref, cb0_ref, cb1_ref, cb2_ref, cb3_ref,
                xq_ref, loss_ref, idx_ref, dist_ref,
                e2_ref, cb2x_ref, hi_ref, mid_ref, lo_ref):
    i = pl.program_id(0)
    cb_refs = (cb0_ref, cb1_ref, cb2_ref, cb3_ref)

    @pl.when(i == 0)
    def _init():
        loss_ref[...] = jnp.zeros((1, 1), jnp.float32)
        for q, cb_ref in enumerate(cb_refs):
            cb = cb_ref[...]
            e2_ref[q, :] = jnp.sum(cb ** 2, axis=1)
            cb2x_ref[q] = cb + cb
            hi = cb.astype(jnp.bfloat16)
            mid_f = cb - hi.astype(jnp.float32)
            mid = mid_f.astype(jnp.bfloat16)
            lo = (mid_f - mid.astype(jnp.float32)).astype(jnp.bfloat16)
            hi_ref[q] = hi
            mid_ref[q] = mid
            lo_ref[q] = lo

    res = x_ref[...]
    accx = jnp.zeros_like(res)
    idxs = jnp.zeros((TILE, NUM_Q), dtype=jnp.int32)
    col_iota = jax.lax.broadcasted_iota(jnp.int32, (TILE, NUM_Q), 1)
    code_iota = jax.lax.broadcasted_iota(jnp.int32, (TILE, N_E), 1)
    tt_acc = jnp.zeros((TILE, E_DIM), dtype=jnp.float32)

    for q in range(NUM_Q):
        x2 = jnp.sum(res ** 2, axis=1, keepdims=True)
        e2 = e2_ref[q, :]
        mm2 = jax.lax.dot_general(res, cb2x_ref[q], (((1,), (1,)), ((), ())))
        d = x2 + e2[None, :] - mm2
        dist_ref[:, q, :] = d

        idx = jnp.argmin(d, axis=1).astype(jnp.int32)
        idxs = jnp.where(col_iota == q, idx[:, None], idxs)

        oh = (code_iota == idx[:, None]).astype(jnp.bfloat16)
        dn = (((1,), (0,)), ((), ()))
        xq_g = ((jax.lax.dot_general(oh, hi_ref[q], dn,
                                     preferred_element_type=jnp.float32)
                 + jax.lax.dot_general(oh, mid_ref[q], dn,
                                       preferred_element_type=jnp.float32))
                + jax.lax.dot_general(oh, lo_ref[q], dn,
                                      preferred_element_type=jnp.float32))
        t = xq_g - res
        tt_acc = tt_acc + t * t
        xr = res + t
        res = res - xr
        accx = accx + xr

    xq_ref[...] = accx
    idx_ref[...] = idxs
    scale = (1.0 + MU) / (NUM_Q * N_TOK * E_DIM)
    loss_ref[...] = loss_ref[...] + scale * jnp.sum(tt_acc)


def _make_call(interpret=False):
    grid = (N_TOK // TILE,)
    cb_spec = pl.BlockSpec((N_E, E_DIM), lambda i: (0, 0))
    return pl.pallas_call(
        _rvq_kernel,
        grid=grid,
        in_specs=[
            pl.BlockSpec((TILE, E_DIM), lambda i: (i, 0)),
            cb_spec, cb_spec, cb_spec, cb_spec,
        ],
        out_specs=[
            pl.BlockSpec((TILE, E_DIM), lambda i: (i, 0)),
            pl.BlockSpec((1, 1), lambda i: (0, 0)),
            pl.BlockSpec((TILE, NUM_Q), lambda i: (i, 0)),
            pl.BlockSpec((TILE, NUM_Q, N_E), lambda i: (i, 0, 0)),
        ],
        out_shape=[
            jax.ShapeDtypeStruct((N_TOK, E_DIM), jnp.float32),
            jax.ShapeDtypeStruct((1, 1), jnp.float32),
            jax.ShapeDtypeStruct((N_TOK, NUM_Q), jnp.int32),
            jax.ShapeDtypeStruct((N_TOK, NUM_Q, N_E), jnp.float32),
        ],
        scratch_shapes=[
            pltpu.VMEM((NUM_Q, N_E), jnp.float32),
            pltpu.VMEM((NUM_Q, N_E, E_DIM), jnp.float32),
            pltpu.VMEM((NUM_Q, N_E, E_DIM), jnp.bfloat16),
            pltpu.VMEM((NUM_Q, N_E, E_DIM), jnp.bfloat16),
            pltpu.VMEM((NUM_Q, N_E, E_DIM), jnp.bfloat16),
        ],
        interpret=interpret,
    )


def kernel(x, codebook_0, codebook_1, codebook_2, codebook_3):
    out = _make_call()(x, codebook_0, codebook_1, codebook_2, codebook_3)
    x_q, loss, indices, distances = out
    return x_q, loss[0, 0], indices, distances


# speedup vs baseline: 1.2078x; 1.2078x over previous
import jax
import jax.numpy as jnp
from jax.experimental import pallas as pl
from jax.experimental.pallas import tpu as pltpu

N_TOK = 16384
E_DIM = 32
N_E = 1024
NUM_Q = 4
MU = 0.25
TILE = 256


def _rvq_kernel(x_ref, cb0_ref, cb1_ref, cb2_ref, cb3_ref,
                xq_ref, loss_ref, idx_ref, dist_ref,
                e2_ref, cb2x_ref, hml_ref):
    i = pl.program_id(0)
    cb_refs = (cb0_ref, cb1_ref, cb2_ref, cb3_ref)

    @pl.when(i == 0)
    def _init():
        loss_ref[...] = jnp.zeros((1, 1), jnp.float32)
        for q, cb_ref in enumerate(cb_refs):
            cb = cb_ref[...]
            e2_ref[q, :] = jnp.sum(cb ** 2, axis=1)
            cb2x_ref[q] = cb + cb
            hi = cb.astype(jnp.bfloat16)
            mid_f = cb - hi.astype(jnp.float32)
            mid = mid_f.astype(jnp.bfloat16)
            lo = (mid_f - mid.astype(jnp.float32)).astype(jnp.bfloat16)
            hml_ref[q] = jnp.concatenate([hi, mid, lo], axis=1)

    res = x_ref[...]
    accx = jnp.zeros_like(res)
    idxs = jnp.zeros((TILE, NUM_Q), dtype=jnp.int32)
    col_iota = jax.lax.broadcasted_iota(jnp.int32, (TILE, NUM_Q), 1)
    code_iota = jax.lax.broadcasted_iota(jnp.int32, (TILE, N_E), 1)
    tt_acc = jnp.zeros((TILE, E_DIM), dtype=jnp.float32)

    for q in range(NUM_Q):
        x2 = jnp.sum(res ** 2, axis=1, keepdims=True)
        e2 = e2_ref[q, :]
        mm2 = jax.lax.dot_general(res, cb2x_ref[q], (((1,), (1,)), ((), ())))
        d = x2 + e2[None, :] - mm2
        dist_ref[:, q, :] = d

        m = jnp.min(d, axis=1, keepdims=True)
        idx = jnp.min(jnp.where(d == m, code_iota, N_E), axis=1)
        idxs = jnp.where(col_iota == q, idx[:, None], idxs)

        oh = (code_iota == idx[:, None]).astype(jnp.bfloat16)
        dn = (((1,), (0,)), ((), ()))
        g3 = jax.lax.dot_general(oh, hml_ref[q], dn,
                                 preferred_element_type=jnp.float32)
        xq_g = ((g3[:, :E_DIM] + g3[:, E_DIM:2 * E_DIM])
                + g3[:, 2 * E_DIM:])
        t = xq_g - res
        tt_acc = tt_acc + t * t
        xr = res + t
        res = res - xr
        accx = accx + xr

    xq_ref[...] = accx
    idx_ref[...] = idxs
    scale = (1.0 + MU) / (NUM_Q * N_TOK * E_DIM)
    loss_ref[...] = loss_ref[...] + scale * jnp.sum(tt_acc)


def _make_call(interpret=False):
    grid = (N_TOK // TILE,)
    cb_spec = pl.BlockSpec((N_E, E_DIM), lambda i: (0, 0))
    return pl.pallas_call(
        _rvq_kernel,
        grid=grid,
        in_specs=[
            pl.BlockSpec((TILE, E_DIM), lambda i: (i, 0)),
            cb_spec, cb_spec, cb_spec, cb_spec,
        ],
        out_specs=[
            pl.BlockSpec((TILE, E_DIM), lambda i: (i, 0)),
            pl.BlockSpec((1, 1), lambda i: (0, 0)),
            pl.BlockSpec((TILE, NUM_Q), lambda i: (i, 0)),
            pl.BlockSpec((TILE, NUM_Q, N_E), lambda i: (i, 0, 0)),
        ],
        out_shape=[
            jax.ShapeDtypeStruct((N_TOK, E_DIM), jnp.float32),
            jax.ShapeDtypeStruct((1, 1), jnp.float32),
            jax.ShapeDtypeStruct((N_TOK, NUM_Q), jnp.int32),
            jax.ShapeDtypeStruct((N_TOK, NUM_Q, N_E), jnp.float32),
        ],
        scratch_shapes=[
            pltpu.VMEM((NUM_Q, N_E), jnp.float32),
            pltpu.VMEM((NUM_Q, N_E, E_DIM), jnp.float32),
            pltpu.VMEM((NUM_Q, N_E, 3 * E_DIM), jnp.bfloat16),
        ],
        interpret=interpret,
    )


def kernel(x, codebook_0, codebook_1, codebook_2, codebook_3):
    out = _make_call()(x, codebook_0, codebook_1, codebook_2, codebook_3)
    x_q, loss, indices, distances = out
    return x_q, loss[0, 0], indices, distances


# speedup vs baseline: 1.2094x; 1.0013x over previous
import jax
import jax.numpy as jnp
from jax.experimental import pallas as pl
from jax.experimental.pallas import tpu as pltpu

N_TOK = 16384
E_DIM = 32
N_E = 1024
NUM_Q = 4
MU = 0.25
TILE = 256


def _rvq_kernel(x_ref, cb0_ref, cb1_ref, cb2_ref, cb3_ref,
                xq_ref, loss_ref, idx_ref, dist_ref,
                e2_ref, cb2x_ref, hml_ref):
    i = pl.program_id(0)
    cb_refs = (cb0_ref, cb1_ref, cb2_ref, cb3_ref)

    @pl.when(i == 0)
    def _init():
        loss_ref[...] = jnp.zeros((1, 1), jnp.float32)
        for q, cb_ref in enumerate(cb_refs):
            cb = cb_ref[...]
            e2 = jnp.sum(cb ** 2, axis=1)
            e2_ref[q] = jnp.broadcast_to(e2[None, :], (TILE, N_E))
            cb2x_ref[q] = cb + cb
            hi = cb.astype(jnp.bfloat16)
            mid_f = cb - hi.astype(jnp.float32)
            mid = mid_f.astype(jnp.bfloat16)
            lo = (mid_f - mid.astype(jnp.float32)).astype(jnp.bfloat16)
            hml_ref[q] = jnp.concatenate([hi, mid, lo], axis=1)

    res = x_ref[...]
    accx = jnp.zeros_like(res)
    idxs = jnp.zeros((TILE, NUM_Q), dtype=jnp.int32)
    col_iota = jax.lax.broadcasted_iota(jnp.int32, (TILE, NUM_Q), 1)
    code_iota = jax.lax.broadcasted_iota(jnp.int32, (TILE, N_E), 1)
    tt_acc = jnp.zeros((TILE, E_DIM), dtype=jnp.float32)

    for q in range(NUM_Q):
        x2 = jnp.sum(res ** 2, axis=1, keepdims=True)
        mm2 = jax.lax.dot_general(res, cb2x_ref[q], (((1,), (1,)), ((), ())))
        d = x2 + e2_ref[q] - mm2
        dist_ref[:, q, :] = d

        m = jnp.min(d, axis=1, keepdims=True)
        idx = jnp.min(jnp.where(d == m, code_iota, N_E), axis=1)
        idxs = jnp.where(col_iota == q, idx[:, None], idxs)

        oh = (code_iota == idx[:, None]).astype(jnp.bfloat16)
        dn = (((1,), (0,)), ((), ()))
        g3 = jax.lax.dot_general(oh, hml_ref[q], dn,
                                 preferred_element_type=jnp.float32)
        xq_g = ((g3[:, :E_DIM] + g3[:, E_DIM:2 * E_DIM])
                + g3[:, 2 * E_DIM:])
        t = xq_g - res
        tt_acc = tt_acc + t * t
        xr = res + t
        res = res - xr
        accx = accx + xr

    xq_ref[...] = accx
    idx_ref[...] = idxs
    scale = (1.0 + MU) / (NUM_Q * N_TOK * E_DIM)
    loss_ref[...] = loss_ref[...] + scale * jnp.sum(tt_acc)


def _make_call(interpret=False):
    grid = (N_TOK // TILE,)
    cb_spec = pl.BlockSpec((N_E, E_DIM), lambda i: (0, 0))
    return pl.pallas_call(
        _rvq_kernel,
        grid=grid,
        in_specs=[
            pl.BlockSpec((TILE, E_DIM), lambda i: (i, 0)),
            cb_spec, cb_spec, cb_spec, cb_spec,
        ],
        out_specs=[
            pl.BlockSpec((TILE, E_DIM), lambda i: (i, 0)),
            pl.BlockSpec((1, 1), lambda i: (0, 0)),
            pl.BlockSpec((TILE, NUM_Q), lambda i: (i, 0)),
            pl.BlockSpec((TILE, NUM_Q, N_E), lambda i: (i, 0, 0)),
        ],
        out_shape=[
            jax.ShapeDtypeStruct((N_TOK, E_DIM), jnp.float32),
            jax.ShapeDtypeStruct((1, 1), jnp.float32),
            jax.ShapeDtypeStruct((N_TOK, NUM_Q), jnp.int32),
            jax.ShapeDtypeStruct((N_TOK, NUM_Q, N_E), jnp.float32),
        ],
        scratch_shapes=[
            pltpu.VMEM((NUM_Q, TILE, N_E), jnp.float32),
            pltpu.VMEM((NUM_Q, N_E, E_DIM), jnp.float32),
            pltpu.VMEM((NUM_Q, N_E, 3 * E_DIM), jnp.bfloat16),
        ],
        interpret=interpret,
    )


def kernel(x, codebook_0, codebook_1, codebook_2, codebook_3):
    out = _make_call()(x, codebook_0, codebook_1, codebook_2, codebook_3)
    x_q, loss, indices, distances = out
    return x_q, loss[0, 0], indices, distances
